# Initial kernel scaffold; baseline (speedup 1.0000x reference)
import jax
import jax.numpy as jnp
from jax.experimental import pallas as pl


def kernel(x, idx_i, idx_j, Wq, Wk):
    raise NotImplementedError("write your pallas kernel here")


# speedup vs baseline: 1.1060x; 1.1060x over previous
import math

import jax
import jax.numpy as jnp
from jax import lax
from jax.experimental import pallas as pl
from jax.experimental.pallas import tpu as pltpu
from jax.experimental.pallas import tpu_sc as plsc

N = 10000
E = 320000
F = 128

NC = 2
NS = 16
NW = NC * NS
EPW = E // NW
CH = 80
NIT = EPW // CH
GRP = CH // 16


def _mm_body(x_ref, wq_ref, wk_ref, q_ref, k_ref):
    xb = x_ref[...]
    scale = 1.0 / math.sqrt(F)
    q_ref[...] = jnp.dot(xb, wq_ref[...], preferred_element_type=jnp.float32) * scale
    k_ref[...] = jnp.dot(xb, wk_ref[...], preferred_element_type=jnp.float32)


def _project(x, Wq, Wk):
    blk = 1000
    return pl.pallas_call(
        _mm_body,
        grid=(N // blk,),
        in_specs=[
            pl.BlockSpec((blk, F), lambda i: (i, 0)),
            pl.BlockSpec((F, F), lambda i: (0, 0)),
            pl.BlockSpec((F, F), lambda i: (0, 0)),
        ],
        out_specs=[
            pl.BlockSpec((blk, F), lambda i: (i, 0)),
            pl.BlockSpec((blk, F), lambda i: (i, 0)),
        ],
        out_shape=[
            jax.ShapeDtypeStruct((N, F), jnp.float32),
            jax.ShapeDtypeStruct((N, F), jnp.float32),
        ],
    )(x, Wq, Wk)


def _sc_body(q_hbm, k_hbm, ii_hbm, jj_hbm, out_hbm,
             ii_v, jj_v, qrows, krows, out_v, sem_q, sem_k):
    c = lax.axis_index("c")
    s = lax.axis_index("s")
    wid = s * NC + c

    pltpu.sync_copy(ii_hbm.at[wid], ii_v)
    pltpu.sync_copy(jj_hbm.at[wid], jj_v)

    def it_body(it, carry):
        cpq = pltpu.async_copy(q_hbm.at[ii_v.at[it]], qrows, sem_q)
        cpk = pltpu.async_copy(k_hbm.at[jj_v.at[it]], krows, sem_k)
        cpq.wait()
        cpk.wait()

        def grp_body(g, carry2):
            rows = g * 16 + lax.iota(jnp.int32, 16)

            def feat(f, acc):
                colf = jnp.full((16,), f, jnp.int32)
                qv = plsc.load_gather(qrows, [rows, colf])
                kv = plsc.load_gather(krows, [rows, colf])
                return acc + qv * kv

            acc = lax.fori_loop(0, F, feat, jnp.zeros((16,), jnp.float32))
            out_v[pl.ds(it * CH + g * 16, 16)] = acc
            return carry2

        lax.fori_loop(0, GRP, grp_body, 0)
        return carry

    lax.fori_loop(0, NIT, it_body, 0)
    pltpu.sync_copy(out_v, out_hbm.at[pl.ds(wid * EPW, EPW)])


def _edge_scores(q, k, ii, jj):
    mesh = plsc.VectorSubcoreMesh(core_axis_name="c", subcore_axis_name="s")
    fn = pl.kernel(
        _sc_body,
        out_type=jax.ShapeDtypeStruct((E,), jnp.float32),
        mesh=mesh,
        scratch_types=[
            pltpu.VMEM((NIT, CH), jnp.int32),
            pltpu.VMEM((NIT, CH), jnp.int32),
            pltpu.VMEM((CH, F), jnp.float32),
            pltpu.VMEM((CH, F), jnp.float32),
            pltpu.VMEM((EPW,), jnp.float32),
            pltpu.SemaphoreType.DMA,
            pltpu.SemaphoreType.DMA,
        ],
        compiler_params=pltpu.CompilerParams(needs_layout_passes=False),
    )
    return fn(q, k, ii, jj)


def kernel(x, idx_i, idx_j, Wq, Wk):
    q, k = _project(x, Wq, Wk)
    ii = idx_i.reshape(NW, NIT, CH)
    jj = idx_j.reshape(NW, NIT, CH)
    return _edge_scores(q, k, ii, jj)


# speedup vs baseline: 1.2532x; 1.1331x over previous
import math

import jax
import jax.numpy as jnp
from jax import lax
from jax.experimental import pallas as pl
from jax.experimental.pallas import tpu as pltpu
from jax.experimental.pallas import tpu_sc as plsc

N = 10000
E = 320000
F = 128

NC = 2
NS = 16
NW = NC * NS
EPW = E // NW
CH = 80
NIT = EPW // CH
GRP = CH // 16


def _mm_body(x_ref, wq_ref, wk_ref, q_ref, k_ref):
    xb = x_ref[...]
    scale = 1.0 / math.sqrt(F)
    q_ref[...] = jnp.dot(xb, wq_ref[...], preferred_element_type=jnp.float32) * scale
    k_ref[...] = jnp.dot(xb, wk_ref[...], preferred_element_type=jnp.float32)


def _project(x, Wq, Wk):
    blk = 1000
    return pl.pallas_call(
        _mm_body,
        grid=(N // blk,),
        in_specs=[
            pl.BlockSpec((blk, F), lambda i: (i, 0)),
            pl.BlockSpec((F, F), lambda i: (0, 0)),
            pl.BlockSpec((F, F), lambda i: (0, 0)),
        ],
        out_specs=[
            pl.BlockSpec((blk, F), lambda i: (i, 0)),
            pl.BlockSpec((blk, F), lambda i: (i, 0)),
        ],
        out_shape=[
            jax.ShapeDtypeStruct((N, F), jnp.float32),
            jax.ShapeDtypeStruct((N, F), jnp.float32),
        ],
    )(x, Wq, Wk)


def _sc_body(q_hbm, k_hbm, ii_hbm, jj_hbm, out_hbm,
             ii_v, jj_v, qr0, kr0, qr1, kr1, out_v,
             sq0, sk0, sq1, sk1):
    c = lax.axis_index("c")
    s = lax.axis_index("s")
    wid = s * NC + c

    pltpu.sync_copy(ii_hbm.at[wid], ii_v)
    pltpu.sync_copy(jj_hbm.at[wid], jj_v)

    bufs = ((qr0, kr0, sq0, sk0), (qr1, kr1, sq1, sk1))

    def start(it, b):
        qr, kr, sq, sk = bufs[b]
        pltpu.async_copy(q_hbm.at[ii_v.at[it]], qr, sq)
        pltpu.async_copy(k_hbm.at[jj_v.at[it]], kr, sk)

    def wait(it, b):
        qr, kr, sq, sk = bufs[b]
        pltpu.make_async_copy(q_hbm.at[ii_v.at[it]], qr, sq).wait()
        pltpu.make_async_copy(k_hbm.at[jj_v.at[it]], kr, sk).wait()

    def compute(it, b):
        qr, kr, _, _ = bufs[b]

        def grp_body(g, carry2):
            rows = g * 16 + lax.iota(jnp.int32, 16)

            def feat(f, acc):
                colf = jnp.full((16,), f, jnp.int32)
                qv = plsc.load_gather(qr, [rows, colf])
                kv = plsc.load_gather(kr, [rows, colf])
                return acc + qv * kv

            acc = lax.fori_loop(0, F, feat, jnp.zeros((16,), jnp.float32),
                                unroll=16)
            out_v[pl.ds(it * CH + g * 16, 16)] = acc
            return carry2

        lax.fori_loop(0, GRP, grp_body, 0)

    start(0, 0)

    def pair_body(p, carry):
        it0 = p * 2
        start(it0 + 1, 1)
        wait(it0, 0)
        compute(it0, 0)
        it1 = it0 + 1
        start(it1 + 1, 0)
        wait(it1, 1)
        compute(it1, 1)
        return carry

    lax.fori_loop(0, (NIT - 1) // 2, pair_body, 0)
    wait(NIT - 1, 0)
    compute(NIT - 1, 0)

    pltpu.sync_copy(out_v, out_hbm.at[pl.ds(wid * EPW, EPW)])


def _edge_scores(q, k, ii, jj):
    mesh = plsc.VectorSubcoreMesh(core_axis_name="c", subcore_axis_name="s")
    fn = pl.kernel(
        _sc_body,
        out_type=jax.ShapeDtypeStruct((E,), jnp.float32),
        mesh=mesh,
        scratch_types=[
            pltpu.VMEM((NIT, CH), jnp.int32),
            pltpu.VMEM((NIT, CH), jnp.int32),
            pltpu.VMEM((CH, F), jnp.float32),
            pltpu.VMEM((CH, F), jnp.float32),
            pltpu.VMEM((CH, F), jnp.float32),
            pltpu.VMEM((CH, F), jnp.float32),
            pltpu.VMEM((EPW,), jnp.float32),
            pltpu.SemaphoreType.DMA,
            pltpu.SemaphoreType.DMA,
            pltpu.SemaphoreType.DMA,
            pltpu.SemaphoreType.DMA,
        ],
        compiler_params=pltpu.CompilerParams(needs_layout_passes=False),
    )
    return fn(q, k, ii, jj)


def kernel(x, idx_i, idx_j, Wq, Wk):
    q, k = _project(x, Wq, Wk)
    ii = idx_i.reshape(NW, NIT, CH)
    jj = idx_j.reshape(NW, NIT, CH)
    return _edge_scores(q, k, ii, jj)
